# Initial kernel scaffold; baseline (speedup 1.0000x reference)
import functools

import jax
import jax.numpy as jnp
from jax import lax
from jax.experimental import pallas as pl
from jax.experimental.pallas import tpu as pltpu
from jax.experimental.pallas import tpu_sc as plsc

_MAX_ID = 128
_L = 16
_NW = 32


def _sc_partials(emb, ids):
    BF, C, HW = emb.shape
    tiles_per_frame = _NW // BF
    pix = HW // tiles_per_frame
    groups = pix // _L
    acc_rows = C + 1
    seg = _L * _MAX_ID

    mesh = plsc.VectorSubcoreMesh(core_axis_name="c", subcore_axis_name="s")

    @functools.partial(
        pl.kernel,
        out_type=jax.ShapeDtypeStruct((_NW, acc_rows, _MAX_ID), jnp.float32),
        mesh=mesh,
        scratch_types=[
            pltpu.VMEM((pix,), jnp.int32),
            pltpu.VMEM((2, pix), jnp.float32),
            pltpu.VMEM((acc_rows * seg,), jnp.float32),
            pltpu.VMEM((acc_rows, _MAX_ID), jnp.float32),
            pltpu.SemaphoreType.DMA,
            pltpu.SemaphoreType.DMA,
            pltpu.SemaphoreType.DMA,
        ],
    )
    def sc_kernel(emb_hbm, ids_hbm, out_hbm, ids_v, val_v, acc_v, stage_v,
                  sem_ids, sem_a, sem_b):
        wid = lax.axis_index("s") * 2 + lax.axis_index("c")
        frame = wid // tiles_per_frame
        px0 = (wid % tiles_per_frame) * pix

        pltpu.make_async_copy(
            ids_hbm.at[frame, pl.ds(px0, pix)], ids_v, sem_ids).start()
        pltpu.make_async_copy(
            emb_hbm.at[frame, 0, pl.ds(px0, pix)], val_v.at[0], sem_a).start()

        zero = jnp.zeros((_L,), jnp.float32)

        def zero_body(i, carry):
            acc_v[pl.ds(i * _L, _L)] = zero
            return carry

        lax.fori_loop(0, acc_rows * seg // _L, zero_body, 0, unroll=8)

        pltpu.make_async_copy(
            ids_hbm.at[frame, pl.ds(px0, pix)], ids_v, sem_ids).wait()
        lane_off = lax.iota(jnp.int32, _L) * _MAX_ID

        def scale_body(g, carry):
            ids_v[pl.ds(g * _L, _L)] = ids_v[pl.ds(g * _L, _L)] + lane_off
            return carry

        lax.fori_loop(0, groups, scale_body, 0, unroll=8)

        ones = jnp.ones((_L,), jnp.float32)
        sems = (sem_a, sem_b)
        for c in range(C):
            buf = c % 2
            pltpu.make_async_copy(
                emb_hbm.at[frame, c, pl.ds(px0, pix)], val_v.at[buf],
                sems[buf]).wait()
            if c + 1 < C:
                pltpu.make_async_copy(
                    emb_hbm.at[frame, c + 1, pl.ds(px0, pix)],
                    val_v.at[1 - buf], sems[1 - buf]).start()
            acc_c = acc_v.at[pl.ds(c * seg, seg)]
            if c == 0:
                cnt = acc_v.at[pl.ds(C * seg, seg)]

                def body0(g, carry):
                    idx = ids_v[pl.ds(g * _L, _L)]
                    plsc.addupdate_scatter(acc_c, [idx],
                                           val_v[0, pl.ds(g * _L, _L)])
                    plsc.addupdate_scatter(cnt, [idx], ones)
                    return carry

                lax.fori_loop(0, groups, body0, 0, unroll=8)
            else:
                def bodyc(g, carry, acc_c=acc_c, buf=buf):
                    idx = ids_v[pl.ds(g * _L, _L)]
                    plsc.addupdate_scatter(acc_c, [idx],
                                           val_v[buf, pl.ds(g * _L, _L)])
                    return carry

                lax.fori_loop(0, groups, bodyc, 0, unroll=8)

        def red_body(r, carry):
            base = r * seg
            for blk in range(_MAX_ID // _L):
                o = blk * _L
                vs = [acc_v[pl.ds(base + l * _MAX_ID + o, _L)]
                      for l in range(_L)]
                while len(vs) > 1:
                    vs = [vs[i] + vs[i + 1] for i in range(0, len(vs), 2)]
                stage_v[r, pl.ds(o, _L)] = vs[0]
            return carry

        lax.fori_loop(0, acc_rows, red_body, 0)

        pltpu.sync_copy(stage_v, out_hbm.at[wid])

    return sc_kernel(emb, ids)


def _tc_finalize(partials, B, F, C):
    tiles_per_frame = _NW // (B * F)

    def tc_kernel(p_ref, o_ref):
        p = p_ref[...]
        p = p.reshape(B * F, tiles_per_frame, C + 1, _MAX_ID).sum(axis=1)
        sums = p[:, :C, :].reshape(B, F, C, _MAX_ID)
        counts = p[:, C, :].reshape(B, F, _MAX_ID)
        means = sums / jnp.maximum(counts, 1.0)[:, :, None, :]
        idpos = lax.broadcasted_iota(jnp.int32, (B, F, _MAX_ID), 2) > 0
        present = (counts > 0.0) & idpos
        common = present[:, :-1] & present[:, 1:]
        d = means[:, 1:] - means[:, :-1]
        dist = jnp.sum(d * d, axis=2)
        total = jnp.sum(jnp.where(common, dist, 0.0))
        valid = jnp.sum(common.astype(jnp.float32))
        o_ref[0, 0] = jnp.where(valid > 0.0,
                                total / jnp.maximum(valid, 1.0),
                                jnp.float32(0.0))

    return pl.pallas_call(
        tc_kernel,
        out_shape=jax.ShapeDtypeStruct((1, 1), jnp.float32),
        out_specs=pl.BlockSpec(memory_space=pltpu.SMEM),
    )(partials)


def kernel(embeddings, track_ids):
    B, F, C, H, W = embeddings.shape
    emb = embeddings.reshape(B * F, C, H * W)
    ids = track_ids.reshape(B * F, H * W).astype(jnp.int32)
    partials = _sc_partials(emb, ids)
    return _tc_finalize(partials, B, F, C)[0, 0]


# speedup vs baseline: 5.8279x; 5.8279x over previous
import functools

import jax
import jax.numpy as jnp
from jax import lax
from jax.experimental import pallas as pl
from jax.experimental.pallas import tpu as pltpu
from jax.experimental.pallas import tpu_sc as plsc

_MAX_ID = 128
_L = 16
_NW = 32


def _sc_partials(emb, ids):
    BF, C, HW = emb.shape
    tiles_per_frame = _NW // BF
    pix = HW // tiles_per_frame
    groups = pix // _L
    acc_rows = C + 1
    seg = _L * _MAX_ID

    mesh = plsc.VectorSubcoreMesh(core_axis_name="c", subcore_axis_name="s")

    @functools.partial(
        pl.kernel,
        out_type=jax.ShapeDtypeStruct((_NW, acc_rows, _MAX_ID), jnp.float32),
        mesh=mesh,
        compiler_params=pltpu.CompilerParams(needs_layout_passes=False),
        scratch_types=[
            pltpu.VMEM((pix,), jnp.int32),
            pltpu.VMEM((2, pix), jnp.float32),
            pltpu.VMEM((acc_rows * seg,), jnp.float32),
            pltpu.VMEM((acc_rows, _MAX_ID), jnp.float32),
            pltpu.SemaphoreType.DMA,
            pltpu.SemaphoreType.DMA,
            pltpu.SemaphoreType.DMA,
        ],
    )
    def sc_kernel(emb_hbm, ids_hbm, out_hbm, ids_v, val_v, acc_v, stage_v,
                  sem_ids, sem_a, sem_b):
        wid = lax.axis_index("s") * 2 + lax.axis_index("c")
        frame = wid // tiles_per_frame
        px0 = (wid % tiles_per_frame) * pix

        pltpu.make_async_copy(
            ids_hbm.at[frame, pl.ds(px0, pix)], ids_v, sem_ids).start()
        pltpu.make_async_copy(
            emb_hbm.at[frame, 0, pl.ds(px0, pix)], val_v.at[0], sem_a).start()

        zero = jnp.zeros((_L,), jnp.float32)

        def zero_body(i, carry):
            acc_v[pl.ds(i * _L, _L)] = zero
            return carry

        lax.fori_loop(0, acc_rows * seg // _L, zero_body, 0, unroll=8)

        pltpu.make_async_copy(
            ids_hbm.at[frame, pl.ds(px0, pix)], ids_v, sem_ids).wait()
        lane_off = lax.iota(jnp.int32, _L) * _MAX_ID

        def scale_body(g, carry):
            ids_v[pl.ds(g * _L, _L)] = ids_v[pl.ds(g * _L, _L)] + lane_off
            return carry

        lax.fori_loop(0, groups, scale_body, 0, unroll=8)

        ones = jnp.ones((_L,), jnp.float32)
        sems = (sem_a, sem_b)
        for c in range(C):
            buf = c % 2
            pltpu.make_async_copy(
                emb_hbm.at[frame, c, pl.ds(px0, pix)], val_v.at[buf],
                sems[buf]).wait()
            if c + 1 < C:
                pltpu.make_async_copy(
                    emb_hbm.at[frame, c + 1, pl.ds(px0, pix)],
                    val_v.at[1 - buf], sems[1 - buf]).start()
            c_off = jnp.full((_L,), c * seg, jnp.int32)
            if c == 0:
                cnt_off = jnp.full((_L,), C * seg, jnp.int32)

                def body0(g, carry):
                    idx = ids_v[pl.ds(g * _L, _L)]
                    plsc.addupdate_scatter(acc_v, [idx + c_off],
                                           val_v[0, pl.ds(g * _L, _L)])
                    plsc.addupdate_scatter(acc_v, [idx + cnt_off], ones)
                    return carry

                lax.fori_loop(0, groups, body0, 0, unroll=8)
            else:
                def bodyc(g, carry, c_off=c_off, buf=buf):
                    idx = ids_v[pl.ds(g * _L, _L)]
                    plsc.addupdate_scatter(acc_v, [idx + c_off],
                                           val_v[buf, pl.ds(g * _L, _L)])
                    return carry

                lax.fori_loop(0, groups, bodyc, 0, unroll=8)

        def red_body(r, carry):
            base = r * seg
            for blk in range(_MAX_ID // _L):
                o = blk * _L
                vs = [acc_v[pl.ds(base + l * _MAX_ID + o, _L)]
                      for l in range(_L)]
                while len(vs) > 1:
                    vs = [vs[i] + vs[i + 1] for i in range(0, len(vs), 2)]
                stage_v[r, pl.ds(o, _L)] = vs[0]
            return carry

        lax.fori_loop(0, acc_rows, red_body, 0)

        pltpu.sync_copy(stage_v, out_hbm.at[wid])

    return sc_kernel(emb, ids)


def _tc_finalize(partials, B, F, C):
    tiles_per_frame = _NW // (B * F)

    def tc_kernel(p_ref, o_ref):
        p = p_ref[...]
        p = p.reshape(B * F, tiles_per_frame, C + 1, _MAX_ID).sum(axis=1)
        sums = p[:, :C, :].reshape(B, F, C, _MAX_ID)
        counts = p[:, C, :].reshape(B, F, _MAX_ID)
        means = sums / jnp.maximum(counts, 1.0)[:, :, None, :]
        idpos = lax.broadcasted_iota(jnp.int32, (B, F, _MAX_ID), 2) > 0
        present = (counts > 0.0) & idpos
        common = present[:, :-1] & present[:, 1:]
        d = means[:, 1:] - means[:, :-1]
        dist = jnp.sum(d * d, axis=2)
        total = jnp.sum(jnp.where(common, dist, 0.0))
        valid = jnp.sum(common.astype(jnp.float32))
        o_ref[0, 0] = jnp.where(valid > 0.0,
                                total / jnp.maximum(valid, 1.0),
                                jnp.float32(0.0))

    return pl.pallas_call(
        tc_kernel,
        out_shape=jax.ShapeDtypeStruct((1, 1), jnp.float32),
        out_specs=pl.BlockSpec(memory_space=pltpu.SMEM),
    )(partials)


def kernel(embeddings, track_ids):
    B, F, C, H, W = embeddings.shape
    emb = embeddings.reshape(B * F, C, H * W)
    ids = track_ids.reshape(B * F, H * W).astype(jnp.int32)
    partials = _sc_partials(emb, ids)
    return _tc_finalize(partials, B, F, C)[0, 0]


# speedup vs baseline: 12.8153x; 2.1990x over previous
import functools

import jax
import jax.numpy as jnp
from jax import lax
from jax.experimental import pallas as pl
from jax.experimental.pallas import tpu as pltpu
from jax.experimental.pallas import tpu_sc as plsc

_MAX_ID = 128
_L = 16
_NW = 32


def _sc_partials(emb, ids):
    BF, C, HW = emb.shape
    tiles_per_frame = _NW // BF
    pix = HW // tiles_per_frame
    acc_rows = C + 1
    seg = _L * _MAX_ID
    psub = 512
    n_sub = pix // psub
    gsub = psub // _L

    mesh = plsc.VectorSubcoreMesh(core_axis_name="c", subcore_axis_name="s")

    @functools.partial(
        pl.kernel,
        out_type=jax.ShapeDtypeStruct((_NW, acc_rows, _MAX_ID), jnp.float32),
        mesh=mesh,
        compiler_params=pltpu.CompilerParams(needs_layout_passes=False),
        scratch_types=[
            pltpu.VMEM((pix,), jnp.int32),
            pltpu.VMEM((2, C, psub), jnp.float32),
            pltpu.VMEM((acc_rows * seg,), jnp.float32),
            pltpu.VMEM((acc_rows, _MAX_ID), jnp.float32),
            pltpu.SemaphoreType.DMA,
            pltpu.SemaphoreType.DMA,
            pltpu.SemaphoreType.DMA,
        ],
    )
    def sc_kernel(emb_hbm, ids_hbm, out_hbm, ids_v, val_v, acc_v, stage_v,
                  sem_ids, sem_a, sem_b):
        wid = lax.axis_index("s") * 2 + lax.axis_index("c")
        frame = wid // tiles_per_frame
        px0 = (wid % tiles_per_frame) * pix

        pltpu.make_async_copy(
            ids_hbm.at[frame, pl.ds(px0, pix)], ids_v, sem_ids).start()
        sems = (sem_a, sem_b)
        pltpu.make_async_copy(
            emb_hbm.at[frame, :, pl.ds(px0, psub)], val_v.at[0],
            sems[0]).start()
        pltpu.make_async_copy(
            emb_hbm.at[frame, :, pl.ds(px0 + psub, psub)], val_v.at[1],
            sems[1]).start()

        zero = jnp.zeros((_L,), jnp.float32)

        def zero_body(i, carry):
            acc_v[pl.ds(i * _L, _L)] = zero
            return carry

        lax.fori_loop(0, acc_rows * seg // _L, zero_body, 0, unroll=8)
        pltpu.make_async_copy(
            ids_hbm.at[frame, pl.ds(px0, pix)], ids_v, sem_ids).wait()

        lane_off = lax.iota(jnp.int32, _L) * _MAX_ID
        ones = jnp.ones((_L,), jnp.float32)

        def sub_body(i, carry):
            for buf in range(2):
                s = 2 * i + buf
                pltpu.make_async_copy(
                    emb_hbm.at[frame, :, pl.ds(px0 + s * psub, psub)],
                    val_v.at[buf], sems[buf]).wait()

                @plsc.parallel_loop(0, gsub)
                def gbody(g, s=s, buf=buf):
                    idx = ids_v[pl.ds(s * psub + g * _L, _L)] + lane_off
                    for c in range(C):
                        plsc.addupdate_scatter(
                            acc_v, [idx + c * seg],
                            val_v[buf, c, pl.ds(g * _L, _L)])
                    plsc.addupdate_scatter(acc_v, [idx + C * seg], ones)

                @pl.when(s + 2 < n_sub)
                def _prefetch(s=s, buf=buf):
                    pltpu.make_async_copy(
                        emb_hbm.at[frame, :,
                                   pl.ds(px0 + (s + 2) * psub, psub)],
                        val_v.at[buf], sems[buf]).start()
            return carry

        lax.fori_loop(0, n_sub // 2, sub_body, 0)

        def red_body(r, carry):
            base = r * seg
            for blk in range(_MAX_ID // _L):
                o = blk * _L
                vs = [acc_v[pl.ds(base + l * _MAX_ID + o, _L)]
                      for l in range(_L)]
                while len(vs) > 1:
                    vs = [vs[i] + vs[i + 1] for i in range(0, len(vs), 2)]
                stage_v[r, pl.ds(o, _L)] = vs[0]
            return carry

        lax.fori_loop(0, acc_rows, red_body, 0)

        pltpu.sync_copy(stage_v, out_hbm.at[wid])

    return sc_kernel(emb, ids)


def _tc_finalize(partials, B, F, C):
    tiles_per_frame = _NW // (B * F)

    def tc_kernel(p_ref, o_ref):
        p = p_ref[...]
        p = p.reshape(B * F, tiles_per_frame, C + 1, _MAX_ID).sum(axis=1)
        sums = p[:, :C, :].reshape(B, F, C, _MAX_ID)
        counts = p[:, C, :].reshape(B, F, _MAX_ID)
        means = sums / jnp.maximum(counts, 1.0)[:, :, None, :]
        idpos = lax.broadcasted_iota(jnp.int32, (B, F, _MAX_ID), 2) > 0
        present = (counts > 0.0) & idpos
        common = present[:, :-1] & present[:, 1:]
        d = means[:, 1:] - means[:, :-1]
        dist = jnp.sum(d * d, axis=2)
        total = jnp.sum(jnp.where(common, dist, 0.0))
        valid = jnp.sum(common.astype(jnp.float32))
        o_ref[0, 0] = jnp.where(valid > 0.0,
                                total / jnp.maximum(valid, 1.0),
                                jnp.float32(0.0))

    return pl.pallas_call(
        tc_kernel,
        out_shape=jax.ShapeDtypeStruct((1, 1), jnp.float32),
        out_specs=pl.BlockSpec(memory_space=pltpu.SMEM),
    )(partials)


def kernel(embeddings, track_ids):
    B, F, C, H, W = embeddings.shape
    emb = embeddings.reshape(B * F, C, H * W)
    ids = track_ids.reshape(B * F, H * W).astype(jnp.int32)
    partials = _sc_partials(emb, ids)
    return _tc_finalize(partials, B, F, C)[0, 0]


# speedup vs baseline: 19.2134x; 1.4993x over previous
import functools

import jax
import jax.numpy as jnp
from jax import lax
from jax.experimental import pallas as pl
from jax.experimental.pallas import tpu as pltpu
from jax.experimental.pallas import tpu_sc as plsc

_MAX_ID = 128
_L = 16
_NW = 32


def _sc_partials(emb, ids):
    B, F, C, H, W = emb.shape
    BF = B * F
    tiles_per_frame = _NW // BF
    rows = H // tiles_per_frame
    rsub = 2
    psub = rsub * W
    n_sub = rows // rsub
    gsub = psub // _L
    grow = W // _L
    acc_rows = C + 1
    seg = _L * _MAX_ID

    mesh = plsc.VectorSubcoreMesh(core_axis_name="c", subcore_axis_name="s")

    @functools.partial(
        pl.kernel,
        out_type=jax.ShapeDtypeStruct((_NW, acc_rows, _MAX_ID), jnp.float32),
        mesh=mesh,
        compiler_params=pltpu.CompilerParams(needs_layout_passes=False),
        scratch_types=[
            pltpu.VMEM((rows, W), jnp.int32),
            pltpu.VMEM((2, C, rsub, W), jnp.float32),
            pltpu.VMEM((acc_rows * seg,), jnp.float32),
            pltpu.VMEM((acc_rows, _MAX_ID), jnp.float32),
            pltpu.SemaphoreType.DMA,
            pltpu.SemaphoreType.DMA,
            pltpu.SemaphoreType.DMA,
        ],
    )
    def sc_kernel(emb_hbm, ids_hbm, out_hbm, ids_v, val_v, acc_v, stage_v,
                  sem_ids, sem_a, sem_b):
        wid = lax.axis_index("s") * 2 + lax.axis_index("c")
        frame = wid // tiles_per_frame
        b = frame // F
        f = frame % F
        r0 = (wid % tiles_per_frame) * rows

        pltpu.make_async_copy(
            ids_hbm.at[b, f, pl.ds(r0, rows), :], ids_v, sem_ids).start()
        sems = (sem_a, sem_b)
        pltpu.make_async_copy(
            emb_hbm.at[b, f, :, pl.ds(r0, rsub), :], val_v.at[0],
            sems[0]).start()
        pltpu.make_async_copy(
            emb_hbm.at[b, f, :, pl.ds(r0 + rsub, rsub), :], val_v.at[1],
            sems[1]).start()

        zero = jnp.zeros((_L,), jnp.float32)

        def zero_body(i, carry):
            acc_v[pl.ds(i * _L, _L)] = zero
            return carry

        lax.fori_loop(0, acc_rows * seg // _L, zero_body, 0, unroll=8)
        pltpu.make_async_copy(
            ids_hbm.at[b, f, pl.ds(r0, rows), :], ids_v, sem_ids).wait()

        lane_off = lax.iota(jnp.int32, _L) * _MAX_ID
        ones = jnp.ones((_L,), jnp.float32)

        def sub_body(i, carry):
            for buf in range(2):
                s = 2 * i + buf
                pltpu.make_async_copy(
                    emb_hbm.at[b, f, :, pl.ds(r0 + s * rsub, rsub), :],
                    val_v.at[buf], sems[buf]).wait()

                @plsc.parallel_loop(0, gsub)
                def gbody(g, s=s, buf=buf):
                    gr = g // grow
                    col = (g % grow) * _L
                    idx = ids_v[s * rsub + gr, pl.ds(col, _L)] + lane_off
                    for c in range(C):
                        plsc.addupdate_scatter(
                            acc_v, [idx + c * seg],
                            val_v[buf, c, gr, pl.ds(col, _L)])
                    plsc.addupdate_scatter(acc_v, [idx + C * seg], ones)

                @pl.when(s + 2 < n_sub)
                def _prefetch(s=s, buf=buf):
                    pltpu.make_async_copy(
                        emb_hbm.at[b, f, :,
                                   pl.ds(r0 + (s + 2) * rsub, rsub), :],
                        val_v.at[buf], sems[buf]).start()
            return carry

        lax.fori_loop(0, n_sub // 2, sub_body, 0)

        def red_body(r, carry):
            base = r * seg
            for blk in range(_MAX_ID // _L):
                o = blk * _L
                vs = [acc_v[pl.ds(base + l * _MAX_ID + o, _L)]
                      for l in range(_L)]
                while len(vs) > 1:
                    vs = [vs[i] + vs[i + 1] for i in range(0, len(vs), 2)]
                stage_v[r, pl.ds(o, _L)] = vs[0]
            return carry

        lax.fori_loop(0, acc_rows, red_body, 0)

        pltpu.sync_copy(stage_v, out_hbm.at[wid])

    return sc_kernel(emb, ids)


def _tc_finalize(partials, B, F, C):
    tiles_per_frame = _NW // (B * F)

    def tc_kernel(p_ref, o_ref):
        p = p_ref[...]
        p = p.reshape(B * F, tiles_per_frame, C + 1, _MAX_ID).sum(axis=1)
        sums = p[:, :C, :].reshape(B, F, C, _MAX_ID)
        counts = p[:, C, :].reshape(B, F, _MAX_ID)
        means = sums / jnp.maximum(counts, 1.0)[:, :, None, :]
        idpos = lax.broadcasted_iota(jnp.int32, (B, F, _MAX_ID), 2) > 0
        present = (counts > 0.0) & idpos
        common = present[:, :-1] & present[:, 1:]
        d = means[:, 1:] - means[:, :-1]
        dist = jnp.sum(d * d, axis=2)
        total = jnp.sum(jnp.where(common, dist, 0.0))
        valid = jnp.sum(common.astype(jnp.float32))
        o_ref[0, 0] = jnp.where(valid > 0.0,
                                total / jnp.maximum(valid, 1.0),
                                jnp.float32(0.0))

    return pl.pallas_call(
        tc_kernel,
        out_shape=jax.ShapeDtypeStruct((1, 1), jnp.float32),
        out_specs=pl.BlockSpec(memory_space=pltpu.SMEM),
    )(partials)


def kernel(embeddings, track_ids):
    B, F, C, H, W = embeddings.shape
    ids = track_ids.reshape(B, F, H, W).astype(jnp.int32)
    partials = _sc_partials(embeddings, ids)
    return _tc_finalize(partials, B, F, C)[0, 0]
